# baseline (speedup 1.0000x reference)
import jax
import jax.numpy as jnp
from jax import lax
from jax.experimental import pallas as pl
from jax.experimental.pallas import tpu as pltpu
from jax.experimental.pallas import tpu_sc as plsc

_B = 8
_G = 2048
_NS = 64
_GRP = _G // _NS
_LANE = 128
_SUB = _G // _LANE
_ROWS = _B * _G
_N = _B * _G * _G
_NTILES = 32
_SPT = (_B * _NS) // _NTILES
_R = 1024


def _sc_vals(y_hbm, x_hbm, yi_hbm, xi_hbm, f_hbm, vals_hbm,
             yi_v, xi_v, eidx_v, xv_v, f_v, g_v, sem):
    wid = lax.axis_index("s") * 2 + lax.axis_index("c")
    s0 = wid * _SPT
    i0 = lax.rem(s0, _NS)
    b = s0 // _NS
    pltpu.sync_copy(yi_hbm.at[pl.ds(i0, _SPT)], yi_v)
    pltpu.sync_copy(xi_hbm.at[pl.ds(i0, _SPT)], xi_v)
    pltpu.sync_copy(x_hbm.at[pl.ds(s0, _SPT)], xv_v)
    pltpu.sync_copy(f_hbm, f_v)
    eidx_v[...] = (yi_v[...] + b * _G) * _G + xi_v[...]
    pltpu.async_copy(y_hbm.at[eidx_v], g_v, sem).wait()
    g_v[...] = g_v[...] + f_v[...] * xv_v[...]
    pltpu.sync_copy(g_v, vals_hbm.at[pl.ds(s0, _SPT)])


def _copy(y_ref, o_ref):
    o_ref[...] = y_ref[...]


def _patch(o_in, v_ref, xi_ref, o_ref):
    col = (lax.broadcasted_iota(jnp.int32, (1, 1, 1, _SUB, _LANE), 3) * _LANE
           + lax.broadcasted_iota(jnp.int32, (1, 1, 1, _SUB, _LANE), 4))
    xcol = xi_ref[...].reshape(1, _NS, 1, 1, 1)
    val = v_ref[...].reshape(1, _NS, 1, 1, 1)
    o_ref[...] = jnp.where(col == xcol, val, o_in[...])


def kernel(Y, X, y_idx, x_idx, f):
    Y1 = Y.reshape(_N)
    Xf = X.reshape(_B * _NS)
    f_arr = jnp.full((16,), f, jnp.float32)
    mesh = plsc.VectorSubcoreMesh(core_axis_name="c", subcore_axis_name="s")
    vals = pl.kernel(
        _sc_vals,
        out_type=jax.ShapeDtypeStruct((_B * _NS,), jnp.float32),
        mesh=mesh,
        scratch_types=[
            pltpu.VMEM((_SPT,), jnp.int32),
            pltpu.VMEM((_SPT,), jnp.int32),
            pltpu.VMEM((_SPT,), jnp.int32),
            pltpu.VMEM((_SPT,), jnp.float32),
            pltpu.VMEM((16,), jnp.float32),
            pltpu.VMEM((_SPT,), jnp.float32),
            pltpu.SemaphoreType.DMA,
        ],
    )(Y1, Xf, y_idx, x_idx, f_arr)

    cp = pl.pallas_call(
        _copy,
        grid=(_ROWS // _R,),
        in_specs=[pl.BlockSpec((_R, _G), lambda j: (j, 0))],
        out_specs=pl.BlockSpec((_R, _G), lambda j: (j, 0)),
        out_shape=jax.ShapeDtypeStruct((_ROWS, _G), jnp.float32),
    )(Y.reshape(_ROWS, _G))

    blk = (1, _NS, 1, _SUB, _LANE)
    bmap = lambda b: (b, 0, 0, 0, 0)
    out = pl.pallas_call(
        _patch,
        grid=(_B,),
        in_specs=[
            pl.BlockSpec(blk, bmap),
            pl.BlockSpec((1, 1, _NS), lambda b: (b, 0, 0)),
            pl.BlockSpec(memory_space=pltpu.VMEM),
        ],
        out_specs=pl.BlockSpec(blk, bmap),
        out_shape=jax.ShapeDtypeStruct((_B, _NS, _GRP, _SUB, _LANE), jnp.float32),
        input_output_aliases={0: 0},
    )(cp.reshape(_B, _NS, _GRP, _SUB, _LANE), vals.reshape(_B, 1, _NS), x_idx)
    return out.reshape(_B, _G, _G)


# speedup vs baseline: 5.0211x; 5.0211x over previous
import jax
import jax.numpy as jnp
from jax import lax
from jax.experimental import pallas as pl
from jax.experimental.pallas import tpu as pltpu

_B = 8
_G = 2048
_NS = 64
_R = 1024


def _body(y_ref, x_ref, yi_ref, xi_ref, f_ref, o_ref):
    j = pl.program_id(1)
    o_ref[...] = y_ref[...]
    r0 = j * _R
    fval = f_ref[0, 0]
    col = lax.broadcasted_iota(jnp.int32, (1, _G), 1)

    def step(s, carry):
        y = yi_ref[s]
        x = xi_ref[s]
        row = y - r0

        @pl.when((row >= 0) & (row < _R))
        def _():
            v = fval * x_ref[0, 0, s]
            o_ref[0, pl.ds(row, 1), :] += jnp.where(col == x, v, 0.0)

        return carry

    lax.fori_loop(0, _NS, step, 0)


def kernel(Y, X, y_idx, x_idx, f):
    f_arr = jnp.asarray(f, jnp.float32).reshape(1, 1)
    grid = (_B, _G // _R)
    return pl.pallas_call(
        _body,
        grid=grid,
        in_specs=[
            pl.BlockSpec((1, _R, _G), lambda b, j: (b, j, 0)),
            pl.BlockSpec((1, 1, _NS), lambda b, j: (b, 0, 0), memory_space=pltpu.SMEM),
            pl.BlockSpec((_NS,), lambda b, j: (0,), memory_space=pltpu.SMEM),
            pl.BlockSpec((_NS,), lambda b, j: (0,), memory_space=pltpu.SMEM),
            pl.BlockSpec((1, 1), lambda b, j: (0, 0), memory_space=pltpu.SMEM),
        ],
        out_specs=pl.BlockSpec((1, _R, _G), lambda b, j: (b, j, 0)),
        out_shape=jax.ShapeDtypeStruct((_B, _G, _G), jnp.float32),
        compiler_params=pltpu.CompilerParams(
            dimension_semantics=("arbitrary", "arbitrary"),
        ),
    )(Y, X.reshape(_B, 1, _NS), y_idx, x_idx, f_arr)
